# Initial kernel scaffold; baseline (speedup 1.0000x reference)
import jax
import jax.numpy as jnp
from jax.experimental import pallas as pl


def kernel(X, emb_weights, free_term):
    raise NotImplementedError("write your pallas kernel here")


# speedup vs baseline: 126.3935x; 126.3935x over previous
import functools

import jax
import jax.numpy as jnp
from jax import lax
from jax.experimental import pallas as pl
from jax.experimental.pallas import tpu as pltpu
from jax.experimental.pallas import tpu_sc as plsc

B = 16384
F = 26
V = 32
L = 16
NW = 32
BPW = B // NW
GRP = BPW // L

_mesh = plsc.VectorSubcoreMesh(core_axis_name="c", subcore_axis_name="s")


@functools.partial(
    pl.kernel,
    mesh=_mesh,
    out_type=jax.ShapeDtypeStruct((B,), jnp.float32),
    scratch_types=[
        pltpu.VMEM((F * V,), jnp.float32),
        pltpu.VMEM((F, BPW), jnp.int32),
        pltpu.VMEM((BPW,), jnp.float32),
        pltpu.VMEM((L,), jnp.float32),
    ],
    compiler_params=pltpu.CompilerParams(needs_layout_passes=False),
)
def _emb_sum_kernel(xt_hbm, tab_hbm, free_hbm, out_hbm,
                    tab_v, x_v, out_v, free_v):
    wid = lax.axis_index("s") * 2 + lax.axis_index("c")
    base = wid * BPW

    pltpu.sync_copy(tab_hbm, tab_v)
    pltpu.sync_copy(free_hbm, free_v)
    pltpu.sync_copy(xt_hbm.at[wid], x_v)

    free_vec = free_v[...]

    def body(j, _):
        o = j * L
        acc = free_vec
        for f in range(F):
            idx = x_v[f, pl.ds(o, L)] + (f * V)
            acc = acc + plsc.load_gather(tab_v, [idx])
        out_v[pl.ds(o, L)] = acc
        return _

    lax.fori_loop(0, GRP, body, None)
    pltpu.sync_copy(out_v, out_hbm.at[pl.ds(base, BPW)])


def kernel(X, emb_weights, free_term):
    xt = X.reshape(NW, BPW, F).transpose(0, 2, 1)
    tab = emb_weights.reshape(F * V)
    free = jnp.broadcast_to(free_term, (L,))
    return _emb_sum_kernel(xt, tab, free)
